# baseline (speedup 1.0000x reference)
import functools

import jax
import jax.numpy as jnp
from jax import lax
from jax.experimental import pallas as pl
from jax.experimental.pallas import tpu as pltpu
from jax.experimental.pallas import tpu_sc as plsc

DIM = 64
CTX = 200
N_WORKERS = 32
CHUNK = 128
LANES = 16
NBUF = 4
ROW_UNROLL = 2


def _embed_body(idx_hbm, table_hbm, pos_hbm, out_hbm,
                idx_v, pos_v, bufs, semg, semw):
    chunks_per_w = idx_v.shape[0]
    rows_per_w = chunks_per_w * CHUNK
    wid = lax.axis_index("s") * 2 + lax.axis_index("c")
    base = wid * rows_per_w
    pltpu.sync_copy(idx_hbm.at[pl.ds(wid * chunks_per_w, chunks_per_w)], idx_v)
    pltpu.sync_copy(pos_hbm, pos_v.at[pl.ds(0, CTX)])
    pltpu.sync_copy(pos_hbm, pos_v.at[pl.ds(CTX, CTX)])

    def gather(c, b):
        return pltpu.async_copy(table_hbm.at[idx_v.at[c]], bufs.at[b], semg[b])

    def writeback(c, b):
        return pltpu.async_copy(bufs.at[b], out_hbm.at[pl.ds(base + c * CHUNK, CHUNK)],
                                semw[b])

    for b in range(NBUF - 1):
        gather(b, b)

    def outer(g0, carry):
        for b in range(NBUF):
            c = g0 * NBUF + b
            pltpu.make_async_copy(table_hbm.at[idx_v.at[c]], bufs.at[b],
                                  semg[b]).wait()
            phase = lax.rem(c * CHUNK, CTX)

            def add_body(r0, _, _b=b, _phase=phase):
                for u in range(ROW_UNROLL):
                    r = r0 * ROW_UNROLL + u
                    for j in range(DIM // LANES):
                        sl = pl.ds(LANES * j, LANES)
                        bufs[_b, r, sl] = bufs[_b, r, sl] + pos_v[_phase + r, sl]
                return _

            lax.fori_loop(0, CHUNK // ROW_UNROLL, add_body, 0)
            writeback(c, b)
            bprev = (b - 1) % NBUF

            @pl.when(c + NBUF - 1 < chunks_per_w)
            def _():
                @pl.when(c >= 1)
                def _():
                    pltpu.make_async_copy(
                        bufs.at[bprev],
                        out_hbm.at[pl.ds(base + (c - 1) * CHUNK, CHUNK)],
                        semw[bprev]).wait()

                gather(c + NBUF - 1, bprev)

        return carry

    lax.fori_loop(0, chunks_per_w // NBUF, outer, 0)

    for b in range(NBUF):
        c = chunks_per_w - NBUF + b
        pltpu.make_async_copy(bufs.at[b],
                              out_hbm.at[pl.ds(base + c * CHUNK, CHUNK)],
                              semw[b]).wait()


def kernel(input_text, embed_table, pos_table):
    batch, seq = input_text.shape
    n = batch * seq
    assert seq == CTX and n % (N_WORKERS * CHUNK * NBUF) == 0
    rows_per_w = n // N_WORKERS
    chunks_per_w = rows_per_w // CHUNK

    idx = input_text.astype(jnp.int32).reshape(n // CHUNK, CHUNK)

    mesh = plsc.VectorSubcoreMesh(core_axis_name="c", subcore_axis_name="s")
    run = functools.partial(
        pl.kernel,
        out_type=jax.ShapeDtypeStruct((n, DIM), jnp.float32),
        mesh=mesh,
        scratch_types=[
            pltpu.VMEM((chunks_per_w, CHUNK), jnp.int32),
            pltpu.VMEM((2 * CTX, DIM), jnp.float32),
            pltpu.VMEM((NBUF, CHUNK, DIM), jnp.float32),
            [pltpu.SemaphoreType.DMA] * NBUF,
            [pltpu.SemaphoreType.DMA] * NBUF,
        ],
        compiler_params=pltpu.CompilerParams(use_tc_tiling_on_sc=False),
    )(_embed_body)
    out = run(idx, embed_table, pos_table)
    return out.reshape(batch, seq, DIM)


# speedup vs baseline: 1.1920x; 1.1920x over previous
import functools

import jax
import jax.numpy as jnp
from jax import lax
from jax.experimental import pallas as pl
from jax.experimental.pallas import tpu as pltpu
from jax.experimental.pallas import tpu_sc as plsc

DIM = 64
CTX = 200
BLK = 128
NBUF = 4
GROUPS_PER_W = 200
LANES = 16


def _embed_body(idx_hbm, table_hbm, pos_hbm, out5_hbm,
                idx_v, pos_v, bufs, outbufs, drain_v, semi, semg, semw):
    wid = lax.axis_index("s") * 2 + lax.axis_index("c")
    gbase = wid * GROUPS_PER_W
    pltpu.sync_copy(pos_hbm, pos_v)
    iota = lax.iota(jnp.int32, LANES)

    def issue_idx(g, b):
        pltpu.async_copy(idx_hbm.at[gbase + g], idx_v.at[b], semi[b])

    for b in range(NBUF):
        issue_idx(b, b)

    def outer(g0, carry):
        for b in range(NBUF):
            g = g0 * NBUF + b
            bprev = (b - 1) % NBUF

            @pl.when(g < GROUPS_PER_W)
            def _():
                pltpu.make_async_copy(idx_hbm.at[gbase + g], idx_v.at[b],
                                      semi[b]).wait()

                pltpu.async_copy(table_hbm.at[idx_v.at[b]], bufs.at[b],
                                 semg[b])

                @pl.when(g + NBUF < GROUPS_PER_W)
                def _():
                    issue_idx(g + NBUF, b)

            @pl.when(jnp.logical_and(g >= 1, g <= GROUPS_PER_W))
            def _():
                gc = gbase + g - 1
                l = lax.shift_right_logical(gc, 5)
                bt = lax.bitwise_and(gc, 31)
                pltpu.make_async_copy(idx_hbm.at[pl.ds(0, DIM)], drain_v,
                                      semg[bprev]).wait()

                @pl.when(g >= NBUF + 1)
                def _():
                    pltpu.make_async_copy(outbufs.at[bprev],
                                          out5_hbm.at[0, :, 0],
                                          semw[bprev]).wait()

                rows_l = jnp.full((LANES,), l, dtype=jnp.int32)

                @plsc.parallel_loop(0, DIM, unroll=8)
                def _(d):
                    cols = jnp.full((LANES,), d, dtype=jnp.int32)
                    p = plsc.load_gather(pos_v, [rows_l, cols])
                    for bi0 in range(0, BLK, LANES):
                        v = plsc.load_gather(bufs.at[bprev], [iota + bi0, cols])
                        outbufs[bprev, lax.shift_right_logical(d, 3),
                                lax.bitwise_and(d, 7), pl.ds(bi0, LANES)] = v + p

                pltpu.async_copy(outbufs.at[bprev], out5_hbm.at[l, :, bt],
                                 semw[bprev])

        return carry

    lax.fori_loop(0, (GROUPS_PER_W + NBUF) // NBUF, outer, 0)

    for b in range(NBUF):
        pltpu.make_async_copy(outbufs.at[b], out5_hbm.at[0, :, 0],
                              semw[b]).wait()


def kernel(input_text, embed_table, pos_table):
    batch, seq = input_text.shape
    assert seq == CTX and batch == 32 * BLK

    idx_t = (jnp.transpose(input_text, (1, 0))
             .reshape(CTX * 32, BLK).astype(jnp.int32))
    table_lin = lax.optimization_barrier(embed_table.reshape(-1))
    table_lin = table_lin.reshape(embed_table.shape)
    mesh = plsc.VectorSubcoreMesh(core_axis_name="c", subcore_axis_name="s")
    run = functools.partial(
        pl.kernel,
        out_type=jax.ShapeDtypeStruct((CTX, 8, 32, 8, BLK), jnp.float32),
        mesh=mesh,
        scratch_types=[
            pltpu.VMEM((NBUF, BLK), jnp.int32),
            pltpu.VMEM((CTX, DIM), jnp.float32),
            pltpu.VMEM((NBUF, BLK, DIM), jnp.float32),
            pltpu.VMEM((NBUF, 8, 8, BLK), jnp.float32),
            pltpu.VMEM((DIM, BLK), jnp.int32),
            [pltpu.SemaphoreType.DMA] * NBUF,
            [pltpu.SemaphoreType.DMA] * NBUF,
            [pltpu.SemaphoreType.DMA] * NBUF,
        ],
        compiler_params=pltpu.CompilerParams(use_tc_tiling_on_sc=False,
                                             needs_layout_passes=False),
    )(_embed_body)
    out5 = run(idx_t, table_lin, pos_table)
    return out5.transpose((2, 4, 0, 1, 3)).reshape(batch, seq, DIM)
